# speedup vs baseline: 1.0454x; 1.0454x over previous
import functools

import jax
import jax.numpy as jnp
from jax.experimental import pallas as pl
from jax.experimental.pallas import tpu as pltpu

T, D, S = 16384, 2048, 16
T_TILE = 512


def _fused_body(x_ref, g_ref, wc_ref, batt_ref, bstrat_ref, out_ref):
    x = x_ref[...]
    acc = jnp.dot(x, wc_ref[...], preferred_element_type=jnp.float32)
    z = acc[:, :S] + batt_ref[...] + g_ref[...]
    m = jnp.max(z, axis=-1, keepdims=True)
    e = jnp.exp(z - m)
    w = e / jnp.sum(e, axis=-1, keepdims=True)
    sig = acc[:, S:] + bstrat_ref[...]
    out_ref[...] = jnp.sum(w * sig, axis=-1, keepdims=True)


@jax.jit
def kernel(x, gumbel_noise, W_att, b_att, W_strat, b_strat, adaptive_bias):
    Wc = jnp.concatenate([W_att, W_strat[:, :, 0].T], axis=1)
    batt = (b_att + adaptive_bias).reshape(1, S)
    bstrat = b_strat[:, 0].reshape(1, S)
    grid = (T // T_TILE,)
    out = pl.pallas_call(
        _fused_body,
        grid=grid,
        in_specs=[
            pl.BlockSpec((T_TILE, D), lambda i: (i, 0)),
            pl.BlockSpec((T_TILE, S), lambda i: (i, 0)),
            pl.BlockSpec((D, 2 * S), lambda i: (0, 0)),
            pl.BlockSpec((1, S), lambda i: (0, 0)),
            pl.BlockSpec((1, S), lambda i: (0, 0)),
        ],
        out_specs=pl.BlockSpec((T_TILE, 1), lambda i: (i, 0)),
        out_shape=jax.ShapeDtypeStruct((T, 1), jnp.float32),
    )(x, gumbel_noise, Wc, batt, bstrat)
    return out


# speedup vs baseline: 1.1101x; 1.0620x over previous
import functools

import jax
import jax.numpy as jnp
from jax.experimental import pallas as pl
from jax.experimental.pallas import tpu as pltpu

T, D, S = 16384, 2048, 16
T_TILE = 1024


def _fused_body(x_ref, g_ref, wc_ref, batt_ref, bstrat_ref, out_ref):
    x = x_ref[...]
    acc = jnp.dot(x, wc_ref[...], preferred_element_type=jnp.float32)
    z = acc[:, :S] + batt_ref[...] + g_ref[...]
    m = jnp.max(z, axis=-1, keepdims=True)
    e = jnp.exp(z - m)
    w = e / jnp.sum(e, axis=-1, keepdims=True)
    sig = acc[:, S:] + bstrat_ref[...]
    out_ref[...] = jnp.sum(w * sig, axis=-1, keepdims=True)


@jax.jit
def kernel(x, gumbel_noise, W_att, b_att, W_strat, b_strat, adaptive_bias):
    Wc = jnp.concatenate([W_att, W_strat[:, :, 0].T], axis=1)
    batt = (b_att + adaptive_bias).reshape(1, S)
    bstrat = b_strat[:, 0].reshape(1, S)
    grid = (T // T_TILE,)
    out = pl.pallas_call(
        _fused_body,
        grid=grid,
        in_specs=[
            pl.BlockSpec((T_TILE, D), lambda i: (i, 0)),
            pl.BlockSpec((T_TILE, S), lambda i: (i, 0)),
            pl.BlockSpec((D, 2 * S), lambda i: (0, 0)),
            pl.BlockSpec((1, S), lambda i: (0, 0)),
            pl.BlockSpec((1, S), lambda i: (0, 0)),
        ],
        out_specs=pl.BlockSpec((T_TILE, 1), lambda i: (i, 0)),
        out_shape=jax.ShapeDtypeStruct((T, 1), jnp.float32),
    )(x, gumbel_noise, Wc, batt, bstrat)
    return out


# speedup vs baseline: 1.1568x; 1.0420x over previous
import functools

import jax
import jax.numpy as jnp
from jax.experimental import pallas as pl
from jax.experimental.pallas import tpu as pltpu

T, D, S = 16384, 2048, 16
T_TILE = 2048


def _fused_body(x_ref, g_ref, wc_ref, batt_ref, bstrat_ref, out_ref):
    x = x_ref[...]
    acc = jnp.dot(x, wc_ref[...], preferred_element_type=jnp.float32)
    z = acc[:, :S] + batt_ref[...] + g_ref[...]
    m = jnp.max(z, axis=-1, keepdims=True)
    e = jnp.exp(z - m)
    w = e / jnp.sum(e, axis=-1, keepdims=True)
    sig = acc[:, S:] + bstrat_ref[...]
    out_ref[...] = jnp.sum(w * sig, axis=-1, keepdims=True)


@jax.jit
def kernel(x, gumbel_noise, W_att, b_att, W_strat, b_strat, adaptive_bias):
    Wc = jnp.concatenate([W_att, W_strat[:, :, 0].T], axis=1)
    batt = (b_att + adaptive_bias).reshape(1, S)
    bstrat = b_strat[:, 0].reshape(1, S)
    grid = (T // T_TILE,)
    out = pl.pallas_call(
        _fused_body,
        grid=grid,
        in_specs=[
            pl.BlockSpec((T_TILE, D), lambda i: (i, 0)),
            pl.BlockSpec((T_TILE, S), lambda i: (i, 0)),
            pl.BlockSpec((D, 2 * S), lambda i: (0, 0)),
            pl.BlockSpec((1, S), lambda i: (0, 0)),
            pl.BlockSpec((1, S), lambda i: (0, 0)),
        ],
        out_specs=pl.BlockSpec((T_TILE, 1), lambda i: (i, 0)),
        out_shape=jax.ShapeDtypeStruct((T, 1), jnp.float32),
    )(x, gumbel_noise, Wc, batt, bstrat)
    return out


# speedup vs baseline: 1.2143x; 1.0497x over previous
import functools

import jax
import jax.numpy as jnp
from jax.experimental import pallas as pl
from jax.experimental.pallas import tpu as pltpu

T, D, S = 16384, 2048, 16
T_TILE = 1024


def _gate(acc, g, batt, bstrat):
    z = acc[:, :S] + batt + g
    m = jnp.max(z, axis=-1, keepdims=True)
    e = jnp.exp(z - m)
    w = e / jnp.sum(e, axis=-1, keepdims=True)
    sig = acc[:, S:] + bstrat
    return jnp.sum(w * sig, axis=-1, keepdims=True)


def _fused_body(xa_ref, xb_ref, ga_ref, gb_ref, wc_ref, batt_ref, bstrat_ref, out_ref):
    wc = wc_ref[...]
    batt = batt_ref[...]
    bstrat = bstrat_ref[...]
    acc_a = jnp.dot(xa_ref[...], wc, preferred_element_type=jnp.float32)
    out_ref[:T_TILE, :] = _gate(acc_a, ga_ref[...], batt, bstrat)
    acc_b = jnp.dot(xb_ref[...], wc, preferred_element_type=jnp.float32)
    out_ref[T_TILE:, :] = _gate(acc_b, gb_ref[...], batt, bstrat)


@jax.jit
def kernel(x, gumbel_noise, W_att, b_att, W_strat, b_strat, adaptive_bias):
    Wc = jnp.concatenate([W_att, W_strat[:, :, 0].T], axis=1)
    batt = (b_att + adaptive_bias).reshape(1, S)
    bstrat = b_strat[:, 0].reshape(1, S)
    grid = (T // (2 * T_TILE),)
    out = pl.pallas_call(
        _fused_body,
        grid=grid,
        in_specs=[
            pl.BlockSpec((T_TILE, D), lambda i: (2 * i, 0)),
            pl.BlockSpec((T_TILE, D), lambda i: (2 * i + 1, 0)),
            pl.BlockSpec((T_TILE, S), lambda i: (2 * i, 0)),
            pl.BlockSpec((T_TILE, S), lambda i: (2 * i + 1, 0)),
            pl.BlockSpec((D, 2 * S), lambda i: (0, 0)),
            pl.BlockSpec((1, S), lambda i: (0, 0)),
            pl.BlockSpec((1, S), lambda i: (0, 0)),
        ],
        out_specs=pl.BlockSpec((2 * T_TILE, 1), lambda i: (i, 0)),
        out_shape=jax.ShapeDtypeStruct((T, 1), jnp.float32),
    )(x, x, gumbel_noise, gumbel_noise, Wc, batt, bstrat)
    return out


# speedup vs baseline: 1.2709x; 1.0466x over previous
import functools

import jax
import jax.numpy as jnp
from jax.experimental import pallas as pl
from jax.experimental.pallas import tpu as pltpu

T, D, S = 16384, 2048, 16
T_TILE = 512
NSTREAM = 4


def _gate(acc, g, batt, bstrat):
    z = acc[:, :S] + batt + g
    m = jnp.max(z, axis=-1, keepdims=True)
    e = jnp.exp(z - m)
    w = e / jnp.sum(e, axis=-1, keepdims=True)
    sig = acc[:, S:] + bstrat
    return jnp.sum(w * sig, axis=-1, keepdims=True)


def _fused_body(*refs):
    x_refs = refs[:NSTREAM]
    g_refs = refs[NSTREAM:2 * NSTREAM]
    wc_ref, batt_ref, bstrat_ref, out_ref = refs[2 * NSTREAM:]
    wc = wc_ref[...]
    batt = batt_ref[...]
    bstrat = bstrat_ref[...]
    for j in range(NSTREAM):
        acc = jnp.dot(x_refs[j][...], wc, preferred_element_type=jnp.float32)
        out_ref[j * T_TILE:(j + 1) * T_TILE, :] = _gate(acc, g_refs[j][...], batt, bstrat)


@jax.jit
def kernel(x, gumbel_noise, W_att, b_att, W_strat, b_strat, adaptive_bias):
    Wc = jnp.concatenate([W_att, W_strat[:, :, 0].T], axis=1)
    batt = (b_att + adaptive_bias).reshape(1, S)
    bstrat = b_strat[:, 0].reshape(1, S)
    grid = (T // (NSTREAM * T_TILE),)

    def xmap(j):
        return lambda i: (NSTREAM * i + j, 0)

    out = pl.pallas_call(
        _fused_body,
        grid=grid,
        in_specs=(
            [pl.BlockSpec((T_TILE, D), xmap(j)) for j in range(NSTREAM)]
            + [pl.BlockSpec((T_TILE, S), xmap(j)) for j in range(NSTREAM)]
            + [
                pl.BlockSpec((D, 2 * S), lambda i: (0, 0)),
                pl.BlockSpec((1, S), lambda i: (0, 0)),
                pl.BlockSpec((1, S), lambda i: (0, 0)),
            ]
        ),
        out_specs=pl.BlockSpec((NSTREAM * T_TILE, 1), lambda i: (i, 0)),
        out_shape=jax.ShapeDtypeStruct((T, 1), jnp.float32),
    )(*([x] * NSTREAM + [gumbel_noise] * NSTREAM + [Wc, batt, bstrat]))
    return out


# speedup vs baseline: 1.2979x; 1.0213x over previous
import functools

import jax
import jax.numpy as jnp
from jax.experimental import pallas as pl
from jax.experimental.pallas import tpu as pltpu

T, D, S = 16384, 2048, 16
T_TILE = 256
NSTREAM = 8


def _gate(acc, g, batt, bstrat):
    z = acc[:, :S] + batt + g
    m = jnp.max(z, axis=-1, keepdims=True)
    e = jnp.exp(z - m)
    w = e / jnp.sum(e, axis=-1, keepdims=True)
    sig = acc[:, S:] + bstrat
    return jnp.sum(w * sig, axis=-1, keepdims=True)


def _fused_body(*refs):
    x_refs = refs[:NSTREAM]
    g_refs = refs[NSTREAM:2 * NSTREAM]
    wc_ref, batt_ref, bstrat_ref, out_ref = refs[2 * NSTREAM:]
    wc = wc_ref[...]
    batt = batt_ref[...]
    bstrat = bstrat_ref[...]
    for j in range(NSTREAM):
        acc = jnp.dot(x_refs[j][...], wc, preferred_element_type=jnp.float32)
        out_ref[j * T_TILE:(j + 1) * T_TILE, :] = _gate(acc, g_refs[j][...], batt, bstrat)


@jax.jit
def kernel(x, gumbel_noise, W_att, b_att, W_strat, b_strat, adaptive_bias):
    Wc = jnp.concatenate([W_att, W_strat[:, :, 0].T], axis=1)
    batt = (b_att + adaptive_bias).reshape(1, S)
    bstrat = b_strat[:, 0].reshape(1, S)
    grid = (T // (NSTREAM * T_TILE),)

    def xmap(j):
        return lambda i: (NSTREAM * i + j, 0)

    out = pl.pallas_call(
        _fused_body,
        grid=grid,
        in_specs=(
            [pl.BlockSpec((T_TILE, D), xmap(j)) for j in range(NSTREAM)]
            + [pl.BlockSpec((T_TILE, S), xmap(j)) for j in range(NSTREAM)]
            + [
                pl.BlockSpec((D, 2 * S), lambda i: (0, 0)),
                pl.BlockSpec((1, S), lambda i: (0, 0)),
                pl.BlockSpec((1, S), lambda i: (0, 0)),
            ]
        ),
        out_specs=pl.BlockSpec((NSTREAM * T_TILE, 1), lambda i: (i, 0)),
        out_shape=jax.ShapeDtypeStruct((T, 1), jnp.float32),
    )(*([x] * NSTREAM + [gumbel_noise] * NSTREAM + [Wc, batt, bstrat]))
    return out
